# baseline (speedup 1.0000x reference)
import functools

import numpy as np
import jax
import jax.numpy as jnp
from jax import lax
from jax.experimental import pallas as pl
from jax.experimental.pallas import tpu as pltpu
from jax.experimental.pallas import tpu_sc as plsc

_N = 100000
_E = 3200000
_DIN = 128
_H = 24
_ALPHA = 0.1
_THETA = 0.5
_BETA1 = float(np.log(_THETA / 1.0 + 1.0))
_BETA2 = float(np.log(_THETA / 2.0 + 1.0))

_NC = 2
_NS = 16
_CHUNK = 128
_SB = 4
_NBUF = 4
_LAG = 2
_HALF = _N // _NC
_STRIPE = _HALF // _NS
_NTRASH = 512
_ACC_ROWS = _HALF + _NTRASH
_ZSTRIPE = _ACC_ROWS // _NS
_SSE = _SB * _CHUNK
_G = -(-_E // (_NS * _SSE * 2 * _NBUF)) * 2 * _NBUF
_T = _NS * _G
_EPAD = _T * _SSE


_NDBUF = 2 * _NBUF


def _segsum_body(h_hbm, src_hbm, dloc_hbm, zero_hbm, out_hbm,
                 acc, srcb, dlocb, rows,
                 is0, is1, is2, is3, gs0, gs1, gs2, gs3, ss0, ss1, ss2, ss3):
    c = lax.axis_index("c")
    s = lax.axis_index("s")
    lo = c * _HALF
    isem = (is0, is1, is2, is3)
    gsem = (gs0, gs1, gs2, gs3)
    ssem = (ss0, ss1, ss2, ss3)

    pltpu.sync_copy(zero_hbm.at[pl.ds(s * _ZSTRIPE, _ZSTRIPE)],
                    acc.at[pl.ds(s * _ZSTRIPE, _ZSTRIPE)])
    plsc.subcore_barrier()

    def prefetch(g, b, bd):
        t = s * _G + g
        pltpu.async_copy(src_hbm.at[t], srcb.at[b], isem[b])
        pltpu.async_copy(dloc_hbm.at[c, t], dlocb.at[bd], isem[b])

    def fire(g, b, bd):
        t = s * _G + g
        pltpu.make_async_copy(src_hbm.at[t], srcb.at[b], isem[b]).wait()
        pltpu.make_async_copy(dloc_hbm.at[c, t], dlocb.at[bd],
                              isem[b]).wait()
        for j in range(_SB):
            pltpu.async_copy(h_hbm.at[srcb.at[b, j]], rows.at[b, j],
                             gsem[b])

    def scatter(b, bd):
        for j in range(_SB):
            pltpu.make_async_copy(h_hbm.at[srcb.at[b, j]], rows.at[b, j],
                                  gsem[b]).wait()
        for j in range(_SB):
            pltpu.async_copy(rows.at[b, j], acc.at[dlocb.at[bd, j]],
                             ssem[b], add=True)

    def drain_scatter(b, bd):
        for j in range(_SB):
            pltpu.make_async_copy(rows.at[b, j], acc.at[dlocb.at[bd, j]],
                                  ssem[b]).wait()

    def ring(tp, carry):
        g0 = _NDBUF * tp
        for du in range(_NDBUF):
            g = g0 + du
            b = du % _NBUF
            bd = du

            @pl.when(g >= _NBUF)
            def _():
                drain_scatter(b, (du - _NBUF) % _NDBUF)

            @pl.when(g + 1 < _G)
            def _():
                prefetch(g + 1, (du + 1) % _NBUF, (du + 1) % _NDBUF)
            fire(g, b, bd)

            @pl.when(g >= _LAG)
            def _():
                scatter((du - _LAG) % _NBUF, (du - _LAG) % _NDBUF)
        return carry

    prefetch(0, 0, 0)
    lax.fori_loop(0, _G // _NDBUF, ring, 0)
    for g in range(_G - _LAG, _G):
        scatter(g % _NBUF, g % _NDBUF)
    for g in range(_G - _NBUF, _G):
        drain_scatter(g % _NBUF, g % _NDBUF)
    plsc.subcore_barrier()

    pltpu.sync_copy(acc.at[pl.ds(s * _STRIPE, _STRIPE)],
                    out_hbm.at[pl.ds(lo + s * _STRIPE, _STRIPE)])


def _segment_sum(h, src3, dloc3, zeros_acc):
    mesh = plsc.VectorSubcoreMesh(core_axis_name="c", subcore_axis_name="s",
                                  num_cores=_NC, num_subcores=_NS)
    f = pl.kernel(
        _segsum_body,
        out_type=jax.ShapeDtypeStruct((_N, _H), jnp.float32),
        mesh=mesh,
        scratch_types=[
            pltpu.VMEM_SHARED((_ACC_ROWS, _H), jnp.float32),
            pltpu.VMEM((_NBUF, _SB, _CHUNK), jnp.int32),
            pltpu.VMEM((_NDBUF, _SB, _CHUNK), jnp.int32),
            pltpu.VMEM((_NBUF, _SB, _CHUNK, _H), jnp.float32),
        ] + [pltpu.SemaphoreType.DMA] * (3 * _NBUF),
        compiler_params=pltpu.CompilerParams(use_tc_tiling_on_sc=False),
        name="gcn2_segment_sum",
    )
    return f(h, src3, dloc3, zeros_acc)


_BM = 2000


def _dense1_body(x_ref, w_ref, b_ref, o_ref):
    y = jnp.dot(x_ref[...], w_ref[...], preferred_element_type=jnp.float32)
    o_ref[...] = jax.nn.relu(y + b_ref[...])


def _dense1(x, w, b):
    return pl.pallas_call(
        _dense1_body,
        grid=(_N // _BM,),
        in_specs=[
            pl.BlockSpec((_BM, _DIN), lambda i: (i, 0)),
            pl.BlockSpec((_DIN, _H), lambda i: (0, 0)),
            pl.BlockSpec((1, _H), lambda i: (0, 0)),
        ],
        out_specs=pl.BlockSpec((_BM, _H), lambda i: (i, 0)),
        out_shape=jax.ShapeDtypeStruct((_N, _H), jnp.float32),
        name="gcn2_dense_in",
    )(x, w, b)


def _combine_body(beta, agg_ref, x0_ref, wc_ref, o_ref):
    t = (1.0 - _ALPHA) * agg_ref[...] + _ALPHA * x0_ref[...]
    y = (1.0 - beta) * t + beta * jnp.dot(
        t, wc_ref[...], preferred_element_type=jnp.float32)
    o_ref[...] = jax.nn.relu(y)


def _combine(agg, x0, wc, beta):
    return pl.pallas_call(
        functools.partial(_combine_body, beta),
        grid=(_N // _BM,),
        in_specs=[
            pl.BlockSpec((_BM, _H), lambda i: (i, 0)),
            pl.BlockSpec((_BM, _H), lambda i: (i, 0)),
            pl.BlockSpec((_H, _H), lambda i: (0, 0)),
        ],
        out_specs=pl.BlockSpec((_BM, _H), lambda i: (i, 0)),
        out_shape=jax.ShapeDtypeStruct((_N, _H), jnp.float32),
        name="gcn2_combine",
    )(agg, x0, wc)


def _final_body(agg_ref, x0_ref, wc_ref, w1_ref, b1_ref, o_ref):
    t = (1.0 - _ALPHA) * agg_ref[...] + _ALPHA * x0_ref[...]
    u = jax.nn.relu((1.0 - _BETA2) * t + _BETA2 * jnp.dot(
        t, wc_ref[...], preferred_element_type=jnp.float32))
    logits = jnp.dot(u, w1_ref[...], preferred_element_type=jnp.float32)
    logits = logits + b1_ref[...]
    m = jnp.max(logits, axis=-1, keepdims=True)
    e = jnp.exp(logits - m)
    lse = jnp.log(jnp.sum(e, axis=-1, keepdims=True)) + m
    o_ref[...] = logits - lse


def _final(agg, x0, wc, w1, b1):
    return pl.pallas_call(
        _final_body,
        grid=(_N // _BM,),
        in_specs=[
            pl.BlockSpec((_BM, _H), lambda i: (i, 0)),
            pl.BlockSpec((_BM, _H), lambda i: (i, 0)),
            pl.BlockSpec((_H, _H), lambda i: (0, 0)),
            pl.BlockSpec((_H, _H), lambda i: (0, 0)),
            pl.BlockSpec((1, _H), lambda i: (0, 0)),
        ],
        out_specs=pl.BlockSpec((_BM, _H), lambda i: (i, 0)),
        out_shape=jax.ShapeDtypeStruct((_N, _H), jnp.float32),
        name="gcn2_final",
    )(agg, x0, wc, w1, b1)


def kernel(x, adj_t, W0, b0, Wc1, Wc2, W1, b1):
    src = adj_t[0].astype(jnp.int32)
    dst = adj_t[1].astype(jnp.int32)
    pad = _EPAD - _E
    src_p = jnp.concatenate([src, jnp.zeros((pad,), jnp.int32)])
    dst_p = jnp.concatenate([dst, jnp.full((pad,), _N, jnp.int32)])
    trash = jnp.int32(_HALF) + (dst_p & jnp.int32(_NTRASH - 1))
    dlocs = []
    for core in range(_NC):
        local = dst_p - jnp.int32(core * _HALF)
        ok = (local >= 0) & (local < _HALF)
        dlocs.append(jnp.where(ok, local, trash))
    src3 = src_p.reshape(_T, _SB, _CHUNK)
    dloc3 = jnp.stack(dlocs).reshape(_NC, _T, _SB, _CHUNK)
    zeros_acc = jnp.zeros((_ACC_ROWS, _H), jnp.float32)

    h = _dense1(x, W0, b0.reshape(1, _H))
    agg1 = _segment_sum(h, src3, dloc3, zeros_acc)
    h2 = _combine(agg1, h, Wc1, _BETA1)
    agg2 = _segment_sum(h2, src3, dloc3, zeros_acc)
    return _final(agg2, h, Wc2, W1, b1.reshape(1, _H))


# speedup vs baseline: 1.0273x; 1.0273x over previous
import functools

import numpy as np
import jax
import jax.numpy as jnp
from jax import lax
from jax.experimental import pallas as pl
from jax.experimental.pallas import tpu as pltpu
from jax.experimental.pallas import tpu_sc as plsc

_N = 100000
_E = 3200000
_DIN = 128
_H = 24
_ALPHA = 0.1
_THETA = 0.5
_BETA1 = float(np.log(_THETA / 1.0 + 1.0))
_BETA2 = float(np.log(_THETA / 2.0 + 1.0))

_NC = 2
_NS = 16
_CHUNK = 128
_SB = 4
_NBUF = 4
_LAG = 3
_HALF = _N // _NC
_STRIPE = _HALF // _NS
_NTRASH = 512
_ACC_ROWS = _HALF + _NTRASH
_ZSTRIPE = _ACC_ROWS // _NS
_SSE = _SB * _CHUNK
_G = -(-_E // (_NS * _SSE * _NBUF)) * _NBUF
_T = _NS * _G
_EPAD = _T * _SSE


def _segsum_body(h_hbm, sd_hbm, zero_hbm, out_hbm,
                 acc, idx, dloc, rows, *sems):
    c = lax.axis_index("c")
    s = lax.axis_index("s")
    lo = c * _HALF
    half = jnp.int32(_HALF)
    isem = sems[:_NBUF]
    gsem = sems[_NBUF:2 * _NBUF]
    ssem = sems[2 * _NBUF:]

    pltpu.sync_copy(zero_hbm.at[pl.ds(s * _ZSTRIPE, _ZSTRIPE)],
                    acc.at[pl.ds(s * _ZSTRIPE, _ZSTRIPE)])
    plsc.subcore_barrier()

    def prefetch(g, b):
        t = s * _G + g
        pltpu.async_copy(sd_hbm.at[t], idx.at[b], isem[b])

    def fire(g, b):
        t = s * _G + g
        pltpu.make_async_copy(sd_hbm.at[t], idx.at[b], isem[b]).wait()
        for j in range(_SB):
            for k in range(_CHUNK // 16):
                v = idx[b, 1, j, pl.ds(k * 16, 16)]
                local = v - lo
                ok = (local >= 0) & (local < half)
                trash = half + (v & jnp.int32(_NTRASH - 1))
                dloc[b, j, pl.ds(k * 16, 16)] = jnp.where(ok, local, trash)
        for j in range(_SB):
            pltpu.async_copy(h_hbm.at[idx.at[b, 0, j]], rows.at[b, j],
                             gsem[b])

    def scatter(b):
        for j in range(_SB):
            pltpu.make_async_copy(h_hbm.at[idx.at[b, 0, j]], rows.at[b, j],
                                  gsem[b]).wait()
        for j in range(_SB):
            pltpu.async_copy(rows.at[b, j], acc.at[dloc.at[b, j]],
                             ssem[b], add=True)

    def drain_scatter(b):
        for j in range(_SB):
            pltpu.make_async_copy(rows.at[b, j], acc.at[dloc.at[b, j]],
                                  ssem[b]).wait()

    def ring(tp, carry):
        g0 = _NBUF * tp
        for db in range(_NBUF):
            g = g0 + db
            b = db

            @pl.when(g >= _NBUF)
            def _():
                drain_scatter(b)

            @pl.when(g >= _LAG)
            def _():
                scatter((db - _LAG) % _NBUF)

            @pl.when(g + 1 < _G)
            def _():
                prefetch(g + 1, (db + 1) % _NBUF)
            fire(g, b)
        return carry

    prefetch(0, 0)
    lax.fori_loop(0, _G // _NBUF, ring, 0)
    for g in range(_G - _LAG, _G):
        scatter(g % _NBUF)
    for b in range(_NBUF):
        drain_scatter(b)
    plsc.subcore_barrier()

    pltpu.sync_copy(acc.at[pl.ds(s * _STRIPE, _STRIPE)],
                    out_hbm.at[pl.ds(lo + s * _STRIPE, _STRIPE)])


def _segment_sum(h, sd, zeros_acc):
    mesh = plsc.VectorSubcoreMesh(core_axis_name="c", subcore_axis_name="s",
                                  num_cores=_NC, num_subcores=_NS)
    f = pl.kernel(
        _segsum_body,
        out_type=jax.ShapeDtypeStruct((_N, _H), jnp.float32),
        mesh=mesh,
        scratch_types=[
            pltpu.VMEM_SHARED((_ACC_ROWS, _H), jnp.float32),
            pltpu.VMEM((_NBUF, 2, _SB, _CHUNK), jnp.int32),
            pltpu.VMEM((_NBUF, _SB, _CHUNK), jnp.int32),
            pltpu.VMEM((_NBUF, _SB, _CHUNK, _H), jnp.float32),
        ] + [pltpu.SemaphoreType.DMA] * (3 * _NBUF),
        compiler_params=pltpu.CompilerParams(use_tc_tiling_on_sc=False),
        name="gcn2_segment_sum",
    )
    return f(h, sd, zeros_acc)


_BM = 2000


def _dense1_body(x_ref, w_ref, b_ref, o_ref):
    y = jnp.dot(x_ref[...], w_ref[...], preferred_element_type=jnp.float32)
    o_ref[...] = jax.nn.relu(y + b_ref[...])


def _dense1(x, w, b):
    return pl.pallas_call(
        _dense1_body,
        grid=(_N // _BM,),
        in_specs=[
            pl.BlockSpec((_BM, _DIN), lambda i: (i, 0)),
            pl.BlockSpec((_DIN, _H), lambda i: (0, 0)),
            pl.BlockSpec((1, _H), lambda i: (0, 0)),
        ],
        out_specs=pl.BlockSpec((_BM, _H), lambda i: (i, 0)),
        out_shape=jax.ShapeDtypeStruct((_N, _H), jnp.float32),
        name="gcn2_dense_in",
    )(x, w, b)


def _combine_body(beta, agg_ref, x0_ref, wc_ref, o_ref):
    t = (1.0 - _ALPHA) * agg_ref[...] + _ALPHA * x0_ref[...]
    y = (1.0 - beta) * t + beta * jnp.dot(
        t, wc_ref[...], preferred_element_type=jnp.float32)
    o_ref[...] = jax.nn.relu(y)


def _combine(agg, x0, wc, beta):
    return pl.pallas_call(
        functools.partial(_combine_body, beta),
        grid=(_N // _BM,),
        in_specs=[
            pl.BlockSpec((_BM, _H), lambda i: (i, 0)),
            pl.BlockSpec((_BM, _H), lambda i: (i, 0)),
            pl.BlockSpec((_H, _H), lambda i: (0, 0)),
        ],
        out_specs=pl.BlockSpec((_BM, _H), lambda i: (i, 0)),
        out_shape=jax.ShapeDtypeStruct((_N, _H), jnp.float32),
        name="gcn2_combine",
    )(agg, x0, wc)


def _final_body(agg_ref, x0_ref, wc_ref, w1_ref, b1_ref, o_ref):
    t = (1.0 - _ALPHA) * agg_ref[...] + _ALPHA * x0_ref[...]
    u = jax.nn.relu((1.0 - _BETA2) * t + _BETA2 * jnp.dot(
        t, wc_ref[...], preferred_element_type=jnp.float32))
    logits = jnp.dot(u, w1_ref[...], preferred_element_type=jnp.float32)
    logits = logits + b1_ref[...]
    m = jnp.max(logits, axis=-1, keepdims=True)
    e = jnp.exp(logits - m)
    lse = jnp.log(jnp.sum(e, axis=-1, keepdims=True)) + m
    o_ref[...] = logits - lse


def _final(agg, x0, wc, w1, b1):
    return pl.pallas_call(
        _final_body,
        grid=(_N // _BM,),
        in_specs=[
            pl.BlockSpec((_BM, _H), lambda i: (i, 0)),
            pl.BlockSpec((_BM, _H), lambda i: (i, 0)),
            pl.BlockSpec((_H, _H), lambda i: (0, 0)),
            pl.BlockSpec((_H, _H), lambda i: (0, 0)),
            pl.BlockSpec((1, _H), lambda i: (0, 0)),
        ],
        out_specs=pl.BlockSpec((_BM, _H), lambda i: (i, 0)),
        out_shape=jax.ShapeDtypeStruct((_N, _H), jnp.float32),
        name="gcn2_final",
    )(agg, x0, wc, w1, b1)


def kernel(x, adj_t, W0, b0, Wc1, Wc2, W1, b1):
    src = adj_t[0].astype(jnp.int32)
    dst = adj_t[1].astype(jnp.int32)
    pad = _EPAD - _E
    src_p = jnp.concatenate([src, jnp.zeros((pad,), jnp.int32)])
    dst_p = jnp.concatenate([dst, jnp.full((pad,), _N, jnp.int32)])
    sd = jnp.stack([src_p.reshape(_T, _SB, _CHUNK),
                    dst_p.reshape(_T, _SB, _CHUNK)], axis=1)
    zeros_acc = jnp.zeros((_ACC_ROWS, _H), jnp.float32)

    h = _dense1(x, W0, b0.reshape(1, _H))
    agg1 = _segment_sum(h, sd, zeros_acc)
    h2 = _combine(agg1, h, Wc1, _BETA1)
    agg2 = _segment_sum(h2, sd, zeros_acc)
    return _final(agg2, h, Wc2, W1, b1.reshape(1, _H))
